# baseline (speedup 1.0000x reference)
import functools

import jax
import jax.numpy as jnp
from jax.experimental import pallas as pl
from jax.experimental.pallas import tpu as pltpu

N_NODES = 10000
D = 128
E = 320000
N_LAYERS = 2


def _mm_body(x_ref, w_ref, o_ref):
    o_ref[...] = jnp.dot(x_ref[...], w_ref[...],
                         preferred_element_type=jnp.float32)


def _matmul(x, w):
    n, d = x.shape
    k = w.shape[1]
    bn = 2000
    return pl.pallas_call(
        _mm_body,
        grid=(n // bn,),
        in_specs=[pl.BlockSpec((bn, d), lambda i: (i, 0)),
                  pl.BlockSpec((d, k), lambda i: (0, 0))],
        out_specs=pl.BlockSpec((bn, k), lambda i: (i, 0)),
        out_shape=jax.ShapeDtypeStruct((n, k), jnp.float32),
    )(x, w)


def kernel(user_emb, item_emb, g_values, W1, W2, g_row, g_col):
    emb = jnp.concatenate([user_emb, item_emb], axis=0)
    w1cat = jnp.concatenate([W1[:D], W1[D:]], axis=1)
    gv = g_values
    acc = emb
    for layer in range(N_LAYERS):
        ab = _matmul(emb, w1cat)
        a = jnp.take(ab[:, :D], g_row, axis=0)
        b = jnp.take(ab[:, D:], g_col, axis=0)
        t = jnp.take(emb, g_col, axis=0)
        z = jnp.squeeze(jnp.maximum(a + b, 0.0) @ W2, -1)
        drop = jax.nn.sigmoid(z)
        gv = gv * jnp.exp(-drop * float(layer + 1))
        emb = jax.ops.segment_sum(gv[:, None] * t, g_row, num_segments=N_NODES)
        acc = acc + emb
    return acc / float(N_LAYERS + 1)
